# baseline (speedup 1.0000x reference)
import functools

import jax
import jax.numpy as jnp
from jax import lax
from jax.experimental import pallas as pl
from jax.experimental.pallas import tpu as pltpu
from jax.experimental.pallas import tpu_sc as plsc

EMBED = 32
RB = 4
GATHER_SIZES = (128, 128, 128, 128, 128, 128, 32)


@functools.lru_cache(maxsize=None)
def _make_gather(batch: int, hist: int):
    info = plsc.get_sparse_core_info()
    nc, ns = info.num_cores, info.num_subcores
    nw = nc * ns
    chunk_idx = RB * hist
    assert batch % (nw * RB) == 0 and sum(GATHER_SIZES) == chunk_idx
    g_offs = [sum(GATHER_SIZES[:j]) for j in range(len(GATHER_SIZES))]
    rows_per_w = batch // nw
    n_chunks = rows_per_w // RB
    mesh = plsc.VectorSubcoreMesh(core_axis_name="c", subcore_axis_name="s")

    @functools.partial(
        pl.kernel,
        mesh=mesh,
        out_type=jax.ShapeDtypeStruct((batch, hist, EMBED), jnp.float32),
        scratch_types=[
            pltpu.VMEM((chunk_idx,), jnp.int32),
            pltpu.VMEM((chunk_idx, EMBED), jnp.float32),
            pltpu.SemaphoreType.DMA,
        ],
        compiler_params=pltpu.CompilerParams(use_tc_tiling_on_sc=False),
    )
    def gather_kernel(table_hbm, idx_hbm, out_hbm, idx_v, rows_v, sem):
        wid = lax.axis_index("s") * nc + lax.axis_index("c")
        b_base = wid * rows_per_w

        def body(g, carry):
            bb = b_base + g * RB
            off = bb * hist
            pltpu.sync_copy(idx_hbm.at[pl.ds(off, chunk_idx)], idx_v)
            for o, n in zip(g_offs, GATHER_SIZES):
                pltpu.async_copy(
                    table_hbm.at[idx_v.at[pl.ds(o, n)]],
                    rows_v.at[pl.ds(o, n), :],
                    sem,
                )
            for o, n in zip(g_offs, GATHER_SIZES):
                pltpu.make_async_copy(
                    table_hbm.at[idx_v.at[pl.ds(o, n)]],
                    rows_v.at[pl.ds(o, n), :],
                    sem,
                ).wait()
            for r in range(RB):
                pltpu.sync_copy(
                    rows_v.at[pl.ds(r * hist, hist), :],
                    out_hbm.at[bb + r],
                )
            return carry

        lax.fori_loop(0, n_chunks, body, 0)

    return gather_kernel


def kernel(input_ids, weight):
    batch, hist = input_ids.shape
    ids = input_ids.reshape(-1).astype(jnp.int32)
    return _make_gather(batch, hist)(weight, ids)
